# Initial kernel scaffold; baseline (speedup 1.0000x reference)
import jax
import jax.numpy as jnp
from jax.experimental import pallas as pl


def kernel(x, edge_index, batch, W_l, b_l, W_r, W2, b2):
    raise NotImplementedError("write your pallas kernel here")


# speedup vs baseline: 5.8004x; 5.8004x over previous
import functools

import jax
import jax.numpy as jnp
from jax import lax
from jax.experimental import pallas as pl
from jax.experimental.pallas import tpu as pltpu
from jax.experimental.pallas import tpu_sc as plsc

_NC = 2
_NS = 16
_G = 64

_CHUNK = 80
_EBLK = 4000


def _sc_aggregate(x, src, dst, zeros):
    n, d = x.shape
    e = src.shape[0]
    nw = _NC * _NS
    e_per_w = e // nw
    n_chunks = e_per_w // _CHUNK
    np_ = zeros.shape[0]
    rows_per_sub = np_ // _NS

    mesh = plsc.VectorSubcoreMesh(
        core_axis_name="c", subcore_axis_name="s",
        num_cores=_NC, num_subcores=_NS)

    @functools.partial(
        pl.kernel,
        out_type=jax.ShapeDtypeStruct((nw * rows_per_sub, d), jnp.float32),
        mesh=mesh,
        scratch_types=[
            pltpu.VMEM((_CHUNK,), jnp.int32),
            pltpu.VMEM((_CHUNK,), jnp.int32),
            pltpu.VMEM((_CHUNK, d), jnp.float32),
            pltpu.VMEM_SHARED((np_, d), jnp.float32),
            pltpu.SemaphoreType.DMA,
        ],
    )
    def k(x_hbm, src_hbm, dst_hbm, zeros_hbm, out_part,
          sidx, didx, rows, acc, sem):
        cid = lax.axis_index("c")
        sid = lax.axis_index("s")
        w = cid * _NS + sid
        r0 = sid * rows_per_sub
        pltpu.sync_copy(zeros_hbm.at[pl.ds(r0, rows_per_sub)],
                        acc.at[pl.ds(r0, rows_per_sub)])
        plsc.subcore_barrier()

        base_w = w * e_per_w

        def chunk(j, c):
            base = base_w + j * _CHUNK
            pltpu.sync_copy(src_hbm.at[pl.ds(base, _CHUNK)], sidx)
            pltpu.sync_copy(dst_hbm.at[pl.ds(base, _CHUNK)], didx)
            pltpu.async_copy(x_hbm.at[sidx], rows, sem).wait()
            pltpu.sync_copy(rows, acc.at[didx], add=True)
            return c
        lax.fori_loop(0, n_chunks, chunk, 0)

        plsc.subcore_barrier()
        pltpu.sync_copy(acc.at[pl.ds(r0, rows_per_sub)],
                        out_part.at[pl.ds(w * rows_per_sub, rows_per_sub)])

    return k(x, src, dst, zeros)


def _deg_body(dst_ref, out_ref, acc_ref):
    i = pl.program_id(0)
    nb = pl.num_programs(0)

    @pl.when(i == 0)
    def _():
        acc_ref[...] = jnp.zeros_like(acc_ref)

    ids = dst_ref[...]
    hi = ids // 128
    lo = ids - hi * 128
    nhi = acc_ref.shape[0]
    oh_hi = (hi == lax.broadcasted_iota(jnp.int32, (1, nhi), 1)).astype(jnp.float32)
    oh_lo = (lo == lax.broadcasted_iota(jnp.int32, (1, 128), 1)).astype(jnp.float32)
    acc_ref[...] += lax.dot_general(
        oh_hi, oh_lo, (((0,), (0,)), ((), ())),
        preferred_element_type=jnp.float32)

    @pl.when(i == nb - 1)
    def _():
        out_ref[...] = acc_ref[...]


def _tc_degree(dst2, np_):
    e = dst2.shape[0]
    nb = e // _EBLK
    nhi = np_ // 128
    return pl.pallas_call(
        _deg_body,
        grid=(nb,),
        in_specs=[pl.BlockSpec((_EBLK, 1), lambda i: (i, 0))],
        out_specs=pl.BlockSpec((nhi, 128), lambda i: (0, 0)),
        out_shape=jax.ShapeDtypeStruct((nhi, 128), jnp.float32),
        scratch_shapes=[pltpu.VMEM((nhi, 128), jnp.float32)],
    )(dst2)


def _tc_body(x_ref, p0_ref, p1_ref, deg_ref, b_ref,
             wl_ref, wr_ref, bl_ref, w2_ref, b2_ref, out_ref, pool_ref):
    i = pl.program_id(0)
    nb = pl.num_programs(0)

    @pl.when(i == 0)
    def _():
        pool_ref[...] = jnp.full_like(pool_ref, -1.0)

    summed = p0_ref[...] + p1_ref[...]
    mean = summed / jnp.maximum(deg_ref[...], 1.0)
    h = jnp.dot(mean, wl_ref[...], preferred_element_type=jnp.float32)
    h = h + jnp.dot(x_ref[...], wr_ref[...], preferred_element_type=jnp.float32)
    h = jnp.maximum(h + bl_ref[...], 0.0)

    batch = b_ref[...]
    glo = jnp.min(batch)
    ghi = jnp.max(batch)
    gids = lax.broadcasted_iota(jnp.int32, (_G, 1), 0)

    def body(g, c):
        m = batch == g
        contrib = jnp.max(jnp.where(m, h, -1.0), axis=0)
        upd = jnp.where(gids == g, contrib[None, :], -1.0)
        pool_ref[...] = jnp.maximum(pool_ref[...], upd)
        return c
    lax.fori_loop(glo, ghi + 1, body, 0)

    @pl.when(i == nb - 1)
    def _():
        pooled = jnp.maximum(pool_ref[...], 0.0)
        logits = jnp.dot(pooled, w2_ref[...],
                         preferred_element_type=jnp.float32) + b2_ref[...]
        mx = jnp.max(logits, axis=1, keepdims=True)
        lse = jnp.log(jnp.sum(jnp.exp(logits - mx), axis=1, keepdims=True)) + mx
        out_ref[...] = logits - lse


def _tc_head(x, p0, p1, deg, batch2, wlT, wrT, bl, w2T, b2):
    n, d = x.shape
    h = wlT.shape[1]
    c = w2T.shape[1]
    blk = 1000
    nb = n // blk
    return pl.pallas_call(
        _tc_body,
        grid=(nb,),
        in_specs=[
            pl.BlockSpec((blk, d), lambda i: (i, 0)),
            pl.BlockSpec((blk, d), lambda i: (i, 0)),
            pl.BlockSpec((blk, d), lambda i: (i, 0)),
            pl.BlockSpec((blk, 1), lambda i: (i, 0)),
            pl.BlockSpec((blk, 1), lambda i: (i, 0)),
            pl.BlockSpec((d, h), lambda i: (0, 0)),
            pl.BlockSpec((d, h), lambda i: (0, 0)),
            pl.BlockSpec((1, h), lambda i: (0, 0)),
            pl.BlockSpec((h, c), lambda i: (0, 0)),
            pl.BlockSpec((1, c), lambda i: (0, 0)),
        ],
        out_specs=pl.BlockSpec((_G, c), lambda i: (0, 0)),
        out_shape=jax.ShapeDtypeStruct((_G, c), jnp.float32),
        scratch_shapes=[pltpu.VMEM((_G, h), jnp.float32)],
    )(x, p0, p1, deg, batch2, wlT, wrT, bl, w2T, b2)


def kernel(x, edge_index, batch, W_l, b_l, W_r, W2, b2):
    n, d = x.shape
    h = W_l.shape[0]
    c = W2.shape[0]
    src = edge_index[0]
    dst = edge_index[1]
    np_ = ((n + 127) // 128) * 128
    zeros = jnp.zeros((np_, d), jnp.float32)
    part = _sc_aggregate(x, src, dst, zeros)
    p = part.reshape(_NC, np_, d)[:, :n]
    deg2d = _tc_degree(dst.reshape(-1, 1), np_)
    deg = deg2d.reshape(np_, 1)[:n]
    return _tc_head(x, p[0], p[1], deg, batch.reshape(n, 1),
                    W_l.T, W_r.T, b_l.reshape(1, h), W2.T, b2.reshape(1, c))


# speedup vs baseline: 6.7100x; 1.1568x over previous
import functools

import jax
import jax.numpy as jnp
from jax import lax
from jax.experimental import pallas as pl
from jax.experimental.pallas import tpu as pltpu
from jax.experimental.pallas import tpu_sc as plsc

_NC = 2
_NS = 16
_G = 64

_CHUNK = 80
_NBUF = 4
_EBLK = 4000


def _sc_aggregate(x, src, dst, zeros):
    n, d = x.shape
    e = src.shape[0]
    nw = _NC * _NS
    e_per_w = e // nw
    n_chunks = e_per_w // _CHUNK
    np_ = zeros.shape[0]
    rows_per_sub = np_ // _NS

    mesh = plsc.VectorSubcoreMesh(
        core_axis_name="c", subcore_axis_name="s",
        num_cores=_NC, num_subcores=_NS)

    kk = _NBUF
    n_groups = n_chunks // kk

    @functools.partial(
        pl.kernel,
        out_type=jax.ShapeDtypeStruct((nw * rows_per_sub, d), jnp.float32),
        mesh=mesh,
        scratch_types=(
            [pltpu.VMEM((_CHUNK,), jnp.int32) for _ in range(kk)]
            + [pltpu.VMEM((_CHUNK,), jnp.int32) for _ in range(kk)]
            + [pltpu.VMEM((_CHUNK, d), jnp.float32) for _ in range(kk)]
            + [pltpu.VMEM_SHARED((np_, d), jnp.float32),
               pltpu.SemaphoreType.DMA,
               pltpu.SemaphoreType.DMA,
               pltpu.SemaphoreType.DMA]
        ),
    )
    def k(x_hbm, src_hbm, dst_hbm, zeros_hbm, out_part, *scr):
        sidx = scr[:kk]
        didx = scr[kk:2 * kk]
        rows = scr[2 * kk:3 * kk]
        acc, isem, gsem, ssem = scr[3 * kk:]
        cid = lax.axis_index("c")
        sid = lax.axis_index("s")
        w = cid * _NS + sid
        r0 = sid * rows_per_sub
        pltpu.sync_copy(zeros_hbm.at[pl.ds(r0, rows_per_sub)],
                        acc.at[pl.ds(r0, rows_per_sub)])
        plsc.subcore_barrier()

        base_w = w * e_per_w

        def group(g, c):
            base_g = base_w + g * (kk * _CHUNK)
            dsc = []
            for b in range(kk):
                base = base_g + b * _CHUNK
                dsc.append(pltpu.async_copy(
                    src_hbm.at[pl.ds(base, _CHUNK)], sidx[b], isem))
                dsc.append(pltpu.async_copy(
                    dst_hbm.at[pl.ds(base, _CHUNK)], didx[b], isem))
            for ds_ in dsc:
                ds_.wait()
            dsc = [pltpu.async_copy(x_hbm.at[sidx[b]], rows[b], gsem)
                   for b in range(kk)]
            for ds_ in dsc:
                ds_.wait()
            dsc = [pltpu.async_copy(rows[b], acc.at[didx[b]], ssem, add=True)
                   for b in range(kk)]
            for ds_ in dsc:
                ds_.wait()
            return c
        lax.fori_loop(0, n_groups, group, 0)

        for j in range(n_groups * kk, n_chunks):
            base = base_w + j * _CHUNK
            pltpu.sync_copy(src_hbm.at[pl.ds(base, _CHUNK)], sidx[0])
            pltpu.sync_copy(dst_hbm.at[pl.ds(base, _CHUNK)], didx[0])
            pltpu.async_copy(x_hbm.at[sidx[0]], rows[0], gsem).wait()
            pltpu.async_copy(rows[0], acc.at[didx[0]], ssem, add=True).wait()

        plsc.subcore_barrier()
        pltpu.sync_copy(acc.at[pl.ds(r0, rows_per_sub)],
                        out_part.at[pl.ds(w * rows_per_sub, rows_per_sub)])

    return k(x, src, dst, zeros)


def _deg_body(dst_ref, out_ref, acc_ref):
    i = pl.program_id(0)
    nb = pl.num_programs(0)

    @pl.when(i == 0)
    def _():
        acc_ref[...] = jnp.zeros_like(acc_ref)

    ids = dst_ref[...]
    hi = ids // 128
    lo = ids - hi * 128
    nhi = acc_ref.shape[0]
    oh_hi = (hi == lax.broadcasted_iota(jnp.int32, (1, nhi), 1)).astype(jnp.float32)
    oh_lo = (lo == lax.broadcasted_iota(jnp.int32, (1, 128), 1)).astype(jnp.float32)
    acc_ref[...] += lax.dot_general(
        oh_hi, oh_lo, (((0,), (0,)), ((), ())),
        preferred_element_type=jnp.float32)

    @pl.when(i == nb - 1)
    def _():
        out_ref[...] = acc_ref[...]


def _tc_degree(dst2, np_):
    e = dst2.shape[0]
    nb = e // _EBLK
    nhi = np_ // 128
    return pl.pallas_call(
        _deg_body,
        grid=(nb,),
        in_specs=[pl.BlockSpec((_EBLK, 1), lambda i: (i, 0))],
        out_specs=pl.BlockSpec((nhi, 128), lambda i: (0, 0)),
        out_shape=jax.ShapeDtypeStruct((nhi, 128), jnp.float32),
        scratch_shapes=[pltpu.VMEM((nhi, 128), jnp.float32)],
    )(dst2)


def _tc_body(x_ref, p0_ref, p1_ref, deg_ref, b_ref,
             wl_ref, wr_ref, bl_ref, w2_ref, b2_ref, out_ref, pool_ref):
    i = pl.program_id(0)
    nb = pl.num_programs(0)

    @pl.when(i == 0)
    def _():
        pool_ref[...] = jnp.full_like(pool_ref, -1.0)

    summed = p0_ref[...] + p1_ref[...]
    mean = summed / jnp.maximum(deg_ref[...], 1.0)
    h = jnp.dot(mean, wl_ref[...], preferred_element_type=jnp.float32)
    h = h + jnp.dot(x_ref[...], wr_ref[...], preferred_element_type=jnp.float32)
    h = jnp.maximum(h + bl_ref[...], 0.0)

    batch = b_ref[...]
    glo = jnp.min(batch)
    ghi = jnp.max(batch)
    gids = lax.broadcasted_iota(jnp.int32, (_G, 1), 0)

    def body(g, c):
        m = batch == g
        contrib = jnp.max(jnp.where(m, h, -1.0), axis=0)
        upd = jnp.where(gids == g, contrib[None, :], -1.0)
        pool_ref[...] = jnp.maximum(pool_ref[...], upd)
        return c
    lax.fori_loop(glo, ghi + 1, body, 0)

    @pl.when(i == nb - 1)
    def _():
        pooled = jnp.maximum(pool_ref[...], 0.0)
        logits = jnp.dot(pooled, w2_ref[...],
                         preferred_element_type=jnp.float32) + b2_ref[...]
        mx = jnp.max(logits, axis=1, keepdims=True)
        lse = jnp.log(jnp.sum(jnp.exp(logits - mx), axis=1, keepdims=True)) + mx
        out_ref[...] = logits - lse


def _tc_head(x, p0, p1, deg, batch2, wlT, wrT, bl, w2T, b2):
    n, d = x.shape
    h = wlT.shape[1]
    c = w2T.shape[1]
    blk = 1000
    nb = n // blk
    return pl.pallas_call(
        _tc_body,
        grid=(nb,),
        in_specs=[
            pl.BlockSpec((blk, d), lambda i: (i, 0)),
            pl.BlockSpec((blk, d), lambda i: (i, 0)),
            pl.BlockSpec((blk, d), lambda i: (i, 0)),
            pl.BlockSpec((blk, 1), lambda i: (i, 0)),
            pl.BlockSpec((blk, 1), lambda i: (i, 0)),
            pl.BlockSpec((d, h), lambda i: (0, 0)),
            pl.BlockSpec((d, h), lambda i: (0, 0)),
            pl.BlockSpec((1, h), lambda i: (0, 0)),
            pl.BlockSpec((h, c), lambda i: (0, 0)),
            pl.BlockSpec((1, c), lambda i: (0, 0)),
        ],
        out_specs=pl.BlockSpec((_G, c), lambda i: (0, 0)),
        out_shape=jax.ShapeDtypeStruct((_G, c), jnp.float32),
        scratch_shapes=[pltpu.VMEM((_G, h), jnp.float32)],
    )(x, p0, p1, deg, batch2, wlT, wrT, bl, w2T, b2)


def kernel(x, edge_index, batch, W_l, b_l, W_r, W2, b2):
    n, d = x.shape
    h = W_l.shape[0]
    c = W2.shape[0]
    src = edge_index[0]
    dst = edge_index[1]
    np_ = ((n + 127) // 128) * 128
    zeros = jnp.zeros((np_, d), jnp.float32)
    part = _sc_aggregate(x, src, dst, zeros)
    p = part.reshape(_NC, np_, d)[:, :n]
    deg2d = _tc_degree(dst.reshape(-1, 1), np_)
    deg = deg2d.reshape(np_, 1)[:n]
    return _tc_head(x, p[0], p[1], deg, batch.reshape(n, 1),
                    W_l.T, W_r.T, b_l.reshape(1, h), W2.T, b2.reshape(1, c))
